# baseline (device time: 193883 ns/iter reference)
import jax
import jax.numpy as jnp
import numpy as np
from jax import lax
from jax.experimental import pallas as pl
from jax.experimental.pallas import tpu as pltpu

N_DEV = 4
B_LOC = 2
SQ = 512
D = 1024
H_LOC = 8
DH = 128
SCALE = 0.08838834764831843


def _rope_consts():
    inv = 1.0 / (10000.0 ** (np.arange(0, DH, 2) / DH))
    pos = np.arange(SQ)[:, None] * inv[None, :]
    cos = np.repeat(np.cos(pos), 2, axis=-1).astype(np.float32)
    sin = np.repeat(np.sin(pos), 2, axis=-1).astype(np.float32)
    R = np.zeros((DH, DH), np.float32)
    for k in range(DH // 2):
        R[2 * k + 1, 2 * k] = -1.0
        R[2 * k, 2 * k + 1] = 1.0
    return jnp.asarray(cos), jnp.asarray(sin), jnp.asarray(R)


def kernel(x, Wq, Wk, Wv, Wo):
    cos, sin, R = _rope_consts()
    cos = cos.astype(jnp.bfloat16)
    sin = sin.astype(jnp.bfloat16)
    R = R.astype(jnp.bfloat16)

    def body(x_ref, wq_ref, wk_ref, wv_ref, wo_ref, cos_ref, sin_ref, r_ref,
             out_ref, xsend, xrecv, precv, psend, w16,
             xsend_sems, xrecv_sems, psend_sems, precv_sems):
        my = lax.axis_index("i")

        xsend[...] = x_ref[...].astype(jnp.bfloat16)
        w16[0] = wq_ref[...].astype(jnp.bfloat16)
        w16[1] = wk_ref[...].astype(jnp.bfloat16)
        w16[2] = wv_ref[...].astype(jnp.bfloat16)
        w16[3] = wo_ref[...].astype(jnp.bfloat16)

        barrier_sem = pltpu.get_barrier_semaphore()
        for s in range(N_DEV - 1):
            pl.semaphore_signal(
                barrier_sem, inc=1,
                device_id=((my + 1 + s) % N_DEV,),
                device_id_type=pl.DeviceIdType.MESH,
            )
        pl.semaphore_wait(barrier_sem, N_DEV - 1)

        def rope(t):
            rot = jnp.dot(t, r_ref[...],
                          preferred_element_type=jnp.float32)
            return t * cos_ref[...] + rot.astype(jnp.bfloat16) * sin_ref[...]

        def accumulate_partial(get_x, get_acc, store):
            for b in range(B_LOC):
                xb = get_x(b)
                q = jnp.dot(xb, w16[0], preferred_element_type=jnp.float32)
                k = jnp.dot(xb, w16[1], preferred_element_type=jnp.float32)
                v = jnp.dot(xb, w16[2], preferred_element_type=jnp.float32)
                acc = get_acc(b)
                for h in range(H_LOC):
                    sl = slice(h * DH, (h + 1) * DH)
                    qh = rope(q[:, sl].astype(jnp.bfloat16))
                    kh = rope(k[:, sl].astype(jnp.bfloat16))
                    s = lax.dot_general(
                        qh, kh, (((1,), (1,)), ((), ())),
                        preferred_element_type=jnp.float32,
                    ) * SCALE
                    s = s - jnp.max(s, axis=-1, keepdims=True)
                    w = jnp.exp(s)
                    w = (w / jnp.sum(w, axis=-1, keepdims=True)).astype(
                        jnp.bfloat16)
                    ctx = jnp.dot(w, v[:, sl].astype(jnp.bfloat16),
                                  preferred_element_type=jnp.float32)
                    acc = acc + jnp.dot(ctx.astype(jnp.bfloat16),
                                        w16[3, sl, :],
                                        preferred_element_type=jnp.float32)
                store(b, acc)

        xdesc = [
            pltpu.make_async_remote_copy(
                src_ref=xsend,
                dst_ref=xrecv.at[2 - s],
                send_sem=xsend_sems.at[s],
                recv_sem=xrecv_sems.at[2 - s],
                device_id=((my + 1 + s) % N_DEV,),
                device_id_type=pl.DeviceIdType.MESH,
            )
            for s in range(N_DEV - 1)
        ]
        pdesc = [
            pltpu.make_async_remote_copy(
                src_ref=psend,
                dst_ref=precv.at[2 - s],
                send_sem=psend_sems.at[s],
                recv_sem=precv_sems.at[2 - s],
                device_id=((my + 1 + s) % N_DEV,),
                device_id_type=pl.DeviceIdType.MESH,
            )
            for s in range(N_DEV - 1)
        ]

        xdesc[0].start()
        xdesc[2].start()

        accumulate_partial(
            get_x=lambda b: xsend[b],
            get_acc=lambda b: jnp.zeros((SQ, D), jnp.float32),
            store=lambda b, val: out_ref.__setitem__((b,), val),
        )

        xdesc[1].start()

        for i, s in enumerate((0, 2, 1)):
            xdesc[2 - s].wait_recv()
            if i > 0:
                pdesc[prev].wait_send()
            accumulate_partial(
                get_x=lambda b, _s=s: xrecv[_s, b],
                get_acc=lambda b: jnp.zeros((SQ, D), jnp.float32),
                store=lambda b, val: psend.__setitem__(
                    (b,), val.astype(jnp.bfloat16)),
            )
            pdesc[s].start()
            prev = s

        for k in (2, 0, 1):
            pdesc[2 - k].wait_recv()
            for b in range(B_LOC):
                out_ref[b] = out_ref[b] + precv[k, b].astype(jnp.float32)

        for s in range(N_DEV - 1):
            xdesc[s].wait_send()
        pdesc[prev].wait_send()

    return pl.pallas_call(
        body,
        out_shape=jax.ShapeDtypeStruct((B_LOC, SQ, D), jnp.float32),
        in_specs=[pl.BlockSpec(memory_space=pltpu.VMEM)] * 8,
        out_specs=pl.BlockSpec(memory_space=pltpu.VMEM),
        scratch_shapes=[
            pltpu.VMEM((B_LOC, SQ, D), jnp.bfloat16),
            pltpu.VMEM((N_DEV - 1, B_LOC, SQ, D), jnp.bfloat16),
            pltpu.VMEM((N_DEV - 1, B_LOC, SQ, D), jnp.bfloat16),
            pltpu.VMEM((B_LOC, SQ, D), jnp.bfloat16),
            pltpu.VMEM((4, D, D), jnp.bfloat16),
            pltpu.SemaphoreType.DMA((N_DEV - 1,)),
            pltpu.SemaphoreType.DMA((N_DEV - 1,)),
            pltpu.SemaphoreType.DMA((N_DEV - 1,)),
            pltpu.SemaphoreType.DMA((N_DEV - 1,)),
        ],
        compiler_params=pltpu.CompilerParams(
            collective_id=0, vmem_limit_bytes=100 * 1024 * 1024,
        ),
    )(x, Wq, Wk, Wv, Wo, cos, sin, R)


# device time: 188184 ns/iter; 1.0303x vs baseline; 1.0303x over previous
import jax
import jax.numpy as jnp
import numpy as np
from jax import lax
from jax.experimental import pallas as pl
from jax.experimental.pallas import tpu as pltpu

N_DEV = 4
B_LOC = 2
SQ = 512
D = 1024
H_LOC = 8
DH = 128
SCALE = 0.08838834764831843


def _rope_consts():
    inv = 1.0 / (10000.0 ** (np.arange(0, DH, 2) / DH))
    pos = np.arange(SQ)[:, None] * inv[None, :]
    cos = np.repeat(np.cos(pos), 2, axis=-1).astype(np.float32)
    sin = np.repeat(np.sin(pos), 2, axis=-1).astype(np.float32)
    R = np.zeros((DH, DH), np.float32)
    for k in range(DH // 2):
        R[2 * k + 1, 2 * k] = -1.0
        R[2 * k, 2 * k + 1] = 1.0
    return jnp.asarray(cos), jnp.asarray(sin), jnp.asarray(R)


def kernel(x, Wq, Wk, Wv, Wo):
    cos, sin, R = _rope_consts()

    def body(x_ref, wq_ref, wk_ref, wv_ref, wo_ref, cos_ref, sin_ref, r_ref,
             out_ref, xsend, xrecv, precv, psend,
             xsend_sems, xrecv_sems, psend_sems, precv_sems):
        my = lax.axis_index("i")

        xsend[...] = x_ref[...].astype(jnp.bfloat16)

        barrier_sem = pltpu.get_barrier_semaphore()
        for s in range(N_DEV - 1):
            pl.semaphore_signal(
                barrier_sem, inc=1,
                device_id=((my + 1 + s) % N_DEV,),
                device_id_type=pl.DeviceIdType.MESH,
            )
        pl.semaphore_wait(barrier_sem, N_DEV - 1)

        def rope(t):
            rot = jnp.dot(t, r_ref[...], preferred_element_type=jnp.float32)
            return t * cos_ref[...] + rot * sin_ref[...]

        def accumulate_partial(get_x, get_acc, store):
            for b in range(B_LOC):
                xb = get_x(b)
                q = jnp.dot(xb, wq_ref[...], preferred_element_type=jnp.float32)
                k = jnp.dot(xb, wk_ref[...], preferred_element_type=jnp.float32)
                v = jnp.dot(xb, wv_ref[...], preferred_element_type=jnp.float32)
                acc = get_acc(b)
                for h in range(H_LOC):
                    sl = slice(h * DH, (h + 1) * DH)
                    qh = rope(q[:, sl])
                    kh = rope(k[:, sl])
                    s = lax.dot_general(
                        qh, kh, (((1,), (1,)), ((), ())),
                        preferred_element_type=jnp.float32,
                    ) * SCALE
                    s = s - jnp.max(s, axis=-1, keepdims=True)
                    w = jnp.exp(s)
                    w = w / jnp.sum(w, axis=-1, keepdims=True)
                    ctx = jnp.dot(w, v[:, sl],
                                  preferred_element_type=jnp.float32)
                    acc = acc + jnp.dot(ctx, wo_ref[sl, :],
                                        preferred_element_type=jnp.float32)
                store(b, acc)

        xdesc = [
            pltpu.make_async_remote_copy(
                src_ref=xsend,
                dst_ref=xrecv.at[2 - s],
                send_sem=xsend_sems.at[s],
                recv_sem=xrecv_sems.at[2 - s],
                device_id=((my + 1 + s) % N_DEV,),
                device_id_type=pl.DeviceIdType.MESH,
            )
            for s in range(N_DEV - 1)
        ]
        pdesc = [
            pltpu.make_async_remote_copy(
                src_ref=psend,
                dst_ref=precv.at[2 - s],
                send_sem=psend_sems.at[s],
                recv_sem=precv_sems.at[2 - s],
                device_id=((my + 1 + s) % N_DEV,),
                device_id_type=pl.DeviceIdType.MESH,
            )
            for s in range(N_DEV - 1)
        ]

        xdesc[0].start()
        xdesc[2].start()

        accumulate_partial(
            get_x=lambda b: x_ref[b],
            get_acc=lambda b: jnp.zeros((SQ, D), jnp.float32),
            store=lambda b, val: out_ref.__setitem__((b,), val),
        )

        xdesc[1].start()

        for i, s in enumerate((0, 2, 1)):
            xdesc[2 - s].wait_recv()
            if i > 0:
                pdesc[prev].wait_send()
            accumulate_partial(
                get_x=lambda b, _s=s: xrecv[_s, b].astype(jnp.float32),
                get_acc=lambda b: jnp.zeros((SQ, D), jnp.float32),
                store=lambda b, val: psend.__setitem__(
                    (b,), val.astype(jnp.bfloat16)),
            )
            pdesc[s].start()
            prev = s

        for k in (2, 0, 1):
            pdesc[2 - k].wait_recv()
            for b in range(B_LOC):
                out_ref[b] = out_ref[b] + precv[k, b].astype(jnp.float32)

        for s in range(N_DEV - 1):
            xdesc[s].wait_send()
        pdesc[prev].wait_send()

    return pl.pallas_call(
        body,
        out_shape=jax.ShapeDtypeStruct((B_LOC, SQ, D), jnp.float32),
        in_specs=[pl.BlockSpec(memory_space=pltpu.VMEM)] * 8,
        out_specs=pl.BlockSpec(memory_space=pltpu.VMEM),
        scratch_shapes=[
            pltpu.VMEM((B_LOC, SQ, D), jnp.bfloat16),
            pltpu.VMEM((N_DEV - 1, B_LOC, SQ, D), jnp.bfloat16),
            pltpu.VMEM((N_DEV - 1, B_LOC, SQ, D), jnp.bfloat16),
            pltpu.VMEM((B_LOC, SQ, D), jnp.bfloat16),
            pltpu.SemaphoreType.DMA((N_DEV - 1,)),
            pltpu.SemaphoreType.DMA((N_DEV - 1,)),
            pltpu.SemaphoreType.DMA((N_DEV - 1,)),
            pltpu.SemaphoreType.DMA((N_DEV - 1,)),
        ],
        compiler_params=pltpu.CompilerParams(
            collective_id=0, vmem_limit_bytes=100 * 1024 * 1024,
        ),
    )(x, Wq, Wk, Wv, Wo, cos, sin, R)


# device time: 178492 ns/iter; 1.0862x vs baseline; 1.0543x over previous
import jax
import jax.numpy as jnp
import numpy as np
from jax import lax
from jax.experimental import pallas as pl
from jax.experimental.pallas import tpu as pltpu

N_DEV = 4
B_LOC = 2
SQ = 512
D = 1024
H_LOC = 8
DH = 128
SCALE = 0.08838834764831843


def _rope_consts():
    inv = 1.0 / (10000.0 ** (np.arange(0, DH, 2) / DH))
    pos = np.arange(SQ)[:, None] * inv[None, :]
    cos = np.repeat(np.cos(pos), 2, axis=-1).astype(np.float32)
    sin = np.repeat(np.sin(pos), 2, axis=-1).astype(np.float32)
    R = np.zeros((DH, DH), np.float32)
    for k in range(DH // 2):
        R[2 * k + 1, 2 * k] = -1.0
        R[2 * k, 2 * k + 1] = 1.0
    return jnp.asarray(cos), jnp.asarray(sin), jnp.asarray(R)


def kernel(x, Wq, Wk, Wv, Wo):
    cos, sin, R = _rope_consts()
    qscale = np.float32(SCALE * np.log2(np.e))
    cos_q = cos * qscale
    sin_q = sin * qscale

    def body(x_ref, wq_ref, wk_ref, wv_ref, wo_ref,
             cos_ref, sin_ref, cosq_ref, sinq_ref, r_ref,
             out_ref, xsend, xrecv, precv, psend,
             xsend_sems, xrecv_sems, psend_sems, precv_sems):
        my = lax.axis_index("i")

        xsend[...] = x_ref[...].astype(jnp.bfloat16)

        barrier_sem = pltpu.get_barrier_semaphore()
        for s in range(N_DEV - 1):
            pl.semaphore_signal(
                barrier_sem, inc=1,
                device_id=((my + 1 + s) % N_DEV,),
                device_id_type=pl.DeviceIdType.MESH,
            )
        pl.semaphore_wait(barrier_sem, N_DEV - 1)

        def rope(t, c_ref, s_ref):
            rot = jnp.dot(t, r_ref[...], preferred_element_type=jnp.float32)
            return t * c_ref[...] + rot * s_ref[...]

        def accumulate_partial(get_x, get_acc, store):
            for b in range(B_LOC):
                xb = get_x(b)
                q = jnp.dot(xb, wq_ref[...], preferred_element_type=jnp.float32)
                k = jnp.dot(xb, wk_ref[...], preferred_element_type=jnp.float32)
                v = jnp.dot(xb, wv_ref[...], preferred_element_type=jnp.float32)
                acc = get_acc(b)
                for h in range(H_LOC):
                    sl = slice(h * DH, (h + 1) * DH)
                    qh = rope(q[:, sl], cosq_ref, sinq_ref)
                    kh = rope(k[:, sl], cos_ref, sin_ref)
                    s = lax.dot_general(
                        qh, kh, (((1,), (1,)), ((), ())),
                        preferred_element_type=jnp.float32,
                    )
                    w = jnp.exp2(s)
                    recip = 1.0 / jnp.sum(w, axis=-1, keepdims=True)
                    ctx = jnp.dot(w, v[:, sl],
                                  preferred_element_type=jnp.float32)
                    ctx = ctx * recip
                    acc = acc + jnp.dot(ctx, wo_ref[sl, :],
                                        preferred_element_type=jnp.float32)
                store(b, acc)

        xdesc = [
            pltpu.make_async_remote_copy(
                src_ref=xsend,
                dst_ref=xrecv.at[2 - s],
                send_sem=xsend_sems.at[s],
                recv_sem=xrecv_sems.at[2 - s],
                device_id=((my + 1 + s) % N_DEV,),
                device_id_type=pl.DeviceIdType.MESH,
            )
            for s in range(N_DEV - 1)
        ]
        pdesc = [
            pltpu.make_async_remote_copy(
                src_ref=psend,
                dst_ref=precv.at[2 - s],
                send_sem=psend_sems.at[s],
                recv_sem=precv_sems.at[2 - s],
                device_id=((my + 1 + s) % N_DEV,),
                device_id_type=pl.DeviceIdType.MESH,
            )
            for s in range(N_DEV - 1)
        ]

        xdesc[0].start()
        xdesc[2].start()

        accumulate_partial(
            get_x=lambda b: x_ref[b],
            get_acc=lambda b: jnp.zeros((SQ, D), jnp.float32),
            store=lambda b, val: out_ref.__setitem__((b,), val),
        )

        xdesc[1].start()

        for i, s in enumerate((0, 2, 1)):
            xdesc[2 - s].wait_recv()
            if i > 0:
                pdesc[prev].wait_send()
            accumulate_partial(
                get_x=lambda b, _s=s: xrecv[_s, b].astype(jnp.float32),
                get_acc=lambda b: jnp.zeros((SQ, D), jnp.float32),
                store=lambda b, val: psend.__setitem__(
                    (b,), val.astype(jnp.bfloat16)),
            )
            pdesc[s].start()
            prev = s

        for k in (2, 0, 1):
            pdesc[2 - k].wait_recv()
            for b in range(B_LOC):
                out_ref[b] = out_ref[b] + precv[k, b].astype(jnp.float32)

        for s in range(N_DEV - 1):
            xdesc[s].wait_send()
        pdesc[prev].wait_send()

    return pl.pallas_call(
        body,
        out_shape=jax.ShapeDtypeStruct((B_LOC, SQ, D), jnp.float32),
        in_specs=[pl.BlockSpec(memory_space=pltpu.VMEM)] * 10,
        out_specs=pl.BlockSpec(memory_space=pltpu.VMEM),
        scratch_shapes=[
            pltpu.VMEM((B_LOC, SQ, D), jnp.bfloat16),
            pltpu.VMEM((N_DEV - 1, B_LOC, SQ, D), jnp.bfloat16),
            pltpu.VMEM((N_DEV - 1, B_LOC, SQ, D), jnp.bfloat16),
            pltpu.VMEM((B_LOC, SQ, D), jnp.bfloat16),
            pltpu.SemaphoreType.DMA((N_DEV - 1,)),
            pltpu.SemaphoreType.DMA((N_DEV - 1,)),
            pltpu.SemaphoreType.DMA((N_DEV - 1,)),
            pltpu.SemaphoreType.DMA((N_DEV - 1,)),
        ],
        compiler_params=pltpu.CompilerParams(
            collective_id=0, vmem_limit_bytes=100 * 1024 * 1024,
        ),
    )(x, Wq, Wk, Wv, Wo, cos, sin, cos_q, sin_q, R)
